# baseline (speedup 1.0000x reference)
import functools

import jax
import jax.numpy as jnp
from jax import lax
from jax.experimental import pallas as pl
from jax.experimental.pallas import tpu as pltpu
from jax.experimental.pallas import tpu_sc as plsc

WIN = 500
K = 2
TC_BLOCK = 40
SC_WG = 8
SC_CHUNKS = 32

NEG_INF = float("-inf")


def _tc_fused_body(wt_ref, mx_ref, r_ref, p_ref, idx_ref):
    tc = TC_BLOCK
    mx = mx_ref[...]
    r = r_ref[...]
    inv = 1.0 / WIN
    ms = []
    for j in range(tc):
        m = lax.dot_general(
            mx[:, j, :], r[:, j, :],
            dimension_numbers=(((1,), (1,)), ((), ())),
            preferred_element_type=jnp.float32,
            precision=lax.Precision.HIGHEST,
        )
        ms.append(m * inv)
    M = jnp.stack(ms, axis=0)
    li = lax.broadcasted_iota(jnp.int32, M.shape, 2)
    max1 = jnp.max(M, axis=-1)
    i1 = jnp.min(jnp.where(M == max1[..., None], li, 127), axis=-1)
    M2 = jnp.where(li == i1[..., None], NEG_INF, M)
    max2 = jnp.max(M2, axis=-1)
    i2 = jnp.min(jnp.where(M2 == max2[..., None], li, 127), axis=-1)
    wt0 = wt_ref[0, 0]
    wt1 = wt_ref[1, 0]
    pooled = max1 * wt0 + max2 * wt1
    p_ref[...] = pooled
    idx_ref[...] = jnp.stack([i1, i2], axis=1)


def _tc_panel(mixed3, ref3, weights, bs, n_refs, n_windows):
    grid = n_windows // TC_BLOCK
    out_shape = [
        jax.ShapeDtypeStruct((n_windows, bs), jnp.float32),
        jax.ShapeDtypeStruct((n_windows, K, bs), jnp.int32),
    ]
    in_specs = [
        pl.BlockSpec(memory_space=pltpu.SMEM),
        pl.BlockSpec((bs, TC_BLOCK, WIN), lambda i: (0, i, 0)),
        pl.BlockSpec((n_refs, TC_BLOCK, WIN), lambda i: (0, i, 0)),
    ]
    out_specs = [
        pl.BlockSpec((TC_BLOCK, bs), lambda i: (i, 0)),
        pl.BlockSpec((TC_BLOCK, K, bs), lambda i: (i, 0, 0)),
    ]
    p_t, idx_t = pl.pallas_call(
        _tc_fused_body,
        grid=(grid,),
        in_specs=in_specs,
        out_specs=out_specs,
        out_shape=out_shape,
    )(weights, mixed3, ref3)
    return p_t.T, jnp.transpose(idx_t, (2, 1, 0))


def _sc_dense_body(nw, mx_hbm, r1_hbm, r2_hbm, wts, p1, p2, i1o, i2o,
                   mxbuf, refbuf, accbuf, wtbl, wtbuf, ovbuf, ox1buf, ox2buf):
    n_tasks = nw // SC_WG
    cid = lax.axis_index("c")
    sid = lax.axis_index("s")
    wid = sid * 2 + cid

    @pl.when(wid < n_tasks)
    def _():
        pltpu.sync_copy(wts, wtbuf)
        z16 = jnp.zeros((16,), jnp.float32)
        for b in range(8):
            mxbuf[b, pl.ds(992, 16)] = z16
            mxbuf[b, pl.ds(1008, 16)] = z16
        for p in range(2):
            for rr in range(16):
                refbuf[p, rr, pl.ds(992, 16)] = z16
                refbuf[p, rr, pl.ds(1008, 16)] = z16
        t0 = wid * SC_WG
        ridx = lax.iota(jnp.int32, 16)
        lanes = lax.iota(jnp.int32, 16)

        def pair_step(pair, carry):
            tp = t0 // 2 + pair
            pltpu.sync_copy(mx_hbm.at[:, tp, :], mxbuf.at[:, pl.ds(0, 1000)])
            pltpu.sync_copy(r1_hbm.at[:, tp, :], refbuf.at[0, :, pl.ds(0, 1000)])
            pltpu.sync_copy(r2_hbm.at[:, tp, :], refbuf.at[1, :, pl.ds(0, 1000)])
            for p in range(2):
                for bb in range(4):
                    b0, b1 = 2 * bb, 2 * bb + 1
                    for half in range(2):
                        off = half * 496
                        lo, hi = half * WIN, (half + 1) * WIN

                        def chunk_step(i, accs):
                            base = off + i * 16
                            pos = base + lanes
                            maskf = jnp.where(
                                (pos >= lo) & (pos < hi), 1.0, 0.0
                            ).astype(jnp.float32)
                            m0 = mxbuf[b0, pl.ds(base, 16)] * maskf
                            m1 = mxbuf[b1, pl.ds(base, 16)] * maskf
                            acc0 = list(accs[:16])
                            acc1 = list(accs[16:])
                            for rr in range(16):
                                rc = refbuf[p, rr, pl.ds(base, 16)]
                                acc0[rr] = acc0[rr] + m0 * rc
                                acc1[rr] = acc1[rr] + m1 * rc
                            return tuple(acc0) + tuple(acc1)

                        init = (jnp.zeros((16,), jnp.float32),) * 32
                        accs = lax.fori_loop(0, SC_CHUNKS, chunk_step, init)
                        for j in range(2):
                            for rr in range(16):
                                accbuf[j, rr, :] = accs[16 * j + rr]
                        w = 2 * pair + half
                        for j, b in ((0, b0), (1, b1)):
                            vsum = jnp.zeros((16,), jnp.float32)
                            for l in range(16):
                                col = plsc.load_gather(
                                    accbuf.at[j],
                                    [ridx, jnp.full((16,), l, jnp.int32)],
                                )
                                vsum = vsum + col
                            wtbl[p, b, w, :] = vsum * (1.0 / WIN)
            return carry

        lax.fori_loop(0, SC_WG // 2, pair_step, 0)

        wt0 = wtbuf[0, :]
        wt1 = wtbuf[1, :]
        widx = lax.iota(jnp.int32, 16)
        neg = jnp.full((16,), NEG_INF, jnp.float32)
        zero_i = jnp.zeros((16,), jnp.int32)
        for p in range(2):
            for b in range(8):
                best = plsc.load_gather(wtbl.at[p, b], [widx, zero_i])
                bidx = zero_i
                sec = neg
                sidx = zero_i
                for rr in range(1, 16):
                    v = plsc.load_gather(
                        wtbl.at[p, b], [widx, jnp.full((16,), rr, jnp.int32)]
                    )
                    rvec = jnp.full((16,), rr, jnp.int32)
                    c1 = v > best
                    c2 = v > sec
                    sec = jnp.where(c1, best, jnp.where(c2, v, sec))
                    sidx = jnp.where(c1, bidx, jnp.where(c2, rvec, sidx))
                    best = jnp.where(c1, v, best)
                    bidx = jnp.where(c1, rvec, bidx)
                ovbuf[8 * p + b, :] = best * wt0 + sec * wt1
                ox1buf[8 * p + b, :] = bidx
                ox2buf[8 * p + b, :] = sidx

        for p, (po, io) in enumerate(((p1, i1o), (p2, i2o))):
            src = ovbuf.at[pl.ds(8 * p, 8), pl.ds(0, SC_WG)]
            pltpu.sync_copy(src, po.at[:, pl.ds(t0, SC_WG)])
            pltpu.sync_copy(
                ox1buf.at[pl.ds(8 * p, 8), pl.ds(0, SC_WG)],
                io.at[:, 0, pl.ds(t0, SC_WG)],
            )
            pltpu.sync_copy(
                ox2buf.at[pl.ds(8 * p, 8), pl.ds(0, SC_WG)],
                io.at[:, 1, pl.ds(t0, SC_WG)],
            )


def _sc_panels(mixed3, ref3_1, ref3_2, weights, bs, n_windows):
    mesh = plsc.VectorSubcoreMesh(
        core_axis_name="c", subcore_axis_name="s", num_cores=2, num_subcores=16
    )
    out_type = [
        jax.ShapeDtypeStruct((bs, n_windows), jnp.float32),
        jax.ShapeDtypeStruct((bs, n_windows), jnp.float32),
        jax.ShapeDtypeStruct((bs, K, n_windows), jnp.int32),
        jax.ShapeDtypeStruct((bs, K, n_windows), jnp.int32),
    ]
    scratch = [
        pltpu.VMEM((8, 1024), jnp.float32),
        pltpu.VMEM((2, 16, 1024), jnp.float32),
        pltpu.VMEM((2, 16, 16), jnp.float32),
        pltpu.VMEM((2, 8, 16, 16), jnp.float32),
        pltpu.VMEM((K, 16), jnp.float32),
        pltpu.VMEM((16, 16), jnp.float32),
        pltpu.VMEM((16, 16), jnp.int32),
        pltpu.VMEM((16, 16), jnp.int32),
    ]
    body = functools.partial(_sc_dense_body, n_windows)
    fn = pl.kernel(
        body,
        out_type=out_type,
        mesh=mesh,
        scratch_types=scratch,
        compiler_params=pltpu.CompilerParams(
            needs_layout_passes=False, use_tc_tiling_on_sc=False
        ),
    )
    wts16 = jnp.broadcast_to(weights[:K], (K, 16))
    mixed3p = mixed3.reshape(bs, n_windows // 2, 2 * WIN)
    r1p = ref3_1.reshape(ref3_1.shape[0], n_windows // 2, 2 * WIN)
    r2p = ref3_2.reshape(ref3_2.shape[0], n_windows // 2, 2 * WIN)
    return fn(mixed3p, r1p, r2p, wts16)


def kernel(input_mixed, ref_panel_0, ref_panel_1, ref_panel_2, weights):
    bs, n_snps = input_mixed.shape
    n_refs = ref_panel_0.shape[0]
    n_windows = n_snps // WIN
    mixed3 = input_mixed.reshape(bs, n_windows, WIN)
    r0, r1, r2 = (
        r.reshape(n_refs, n_windows, WIN)
        for r in (ref_panel_0, ref_panel_1, ref_panel_2)
    )
    p0, i0 = _tc_panel(mixed3, r0, weights, bs, n_refs, n_windows)
    p1, p2, i1, i2 = _sc_panels(mixed3, r1, r2, weights, bs, n_windows)
    return (p0, p1, p2, i0, i1, i2)


# speedup vs baseline: 1.5571x; 1.5571x over previous
import functools

import jax
import jax.numpy as jnp
from jax import lax
from jax.experimental import pallas as pl
from jax.experimental.pallas import tpu as pltpu
from jax.experimental.pallas import tpu_sc as plsc

WIN = 500
K = 2
TC_WB = 32
SC_WG = 8
SC_CHUNKS = 32

NEG_INF = float("-inf")


def _tc_fused_body(wt_ref, mx_ref, ra_ref, rb_ref,
                   pa_ref, ia_ref, pb_ref, ib_ref):
    mx = mx_ref[...]
    inv = 1.0 / WIN
    wt0 = wt_ref[0, 0]
    wt1 = wt_ref[1, 0]
    for r_ref, p_ref, idx_ref in ((ra_ref, pa_ref, ia_ref),
                                  (rb_ref, pb_ref, ib_ref)):
        r = r_ref[...]
        ms = []
        for j in range(TC_WB):
            a = lax.slice(mx, (0, j * WIN), (8, (j + 1) * WIN))
            b = lax.slice(r, (0, j * WIN), (16, (j + 1) * WIN))
            m = lax.dot_general(
                a, b,
                dimension_numbers=(((1,), (1,)), ((), ())),
                preferred_element_type=jnp.float32,
                precision=lax.Precision.HIGHEST,
            )
            ms.append(m * inv)
        M = jnp.stack(ms, axis=0)
        li = lax.broadcasted_iota(jnp.int32, M.shape, 2)
        max1 = jnp.max(M, axis=-1)
        i1 = jnp.min(jnp.where(M == max1[..., None], li, 127), axis=-1)
        M2 = jnp.where(li == i1[..., None], NEG_INF, M)
        max2 = jnp.max(M2, axis=-1)
        i2 = jnp.min(jnp.where(M2 == max2[..., None], li, 127), axis=-1)
        p_ref[...] = max1 * wt0 + max2 * wt1
        idx_ref[...] = jnp.stack([i1, i2], axis=1)


def _tc_panels(mixed, ref_a, ref_b, weights, bs, n_refs, n_windows):
    grid = -(-n_windows // TC_WB)
    nw_pad = grid * TC_WB
    cols = TC_WB * WIN
    out_shape = [
        jax.ShapeDtypeStruct((nw_pad, bs), jnp.float32),
        jax.ShapeDtypeStruct((nw_pad, K, bs), jnp.int32),
    ] * 2
    in_specs = [
        pl.BlockSpec(memory_space=pltpu.SMEM),
        pl.BlockSpec((bs, cols), lambda i: (0, i)),
        pl.BlockSpec((n_refs, cols), lambda i: (0, i)),
        pl.BlockSpec((n_refs, cols), lambda i: (0, i)),
    ]
    out_specs = [
        pl.BlockSpec((TC_WB, bs), lambda i: (i, 0)),
        pl.BlockSpec((TC_WB, K, bs), lambda i: (i, 0, 0)),
    ] * 2
    pa, ia, pb, ib = pl.pallas_call(
        _tc_fused_body,
        grid=(grid,),
        in_specs=in_specs,
        out_specs=out_specs,
        out_shape=out_shape,
    )(weights, mixed, ref_a, ref_b)
    outs = []
    for p_t, i_t in ((pa, ia), (pb, ib)):
        outs.append(p_t[:n_windows].T)
        outs.append(jnp.transpose(i_t[:n_windows], (2, 1, 0)))
    return outs


def _sc_dense_body(nw, mx_hbm, r_hbm, wts, po, io,
                   mxbuf, refbuf, accbuf, wtbl, wtbuf, ovbuf, ox1buf, ox2buf):
    n_tasks = nw // SC_WG
    cid = lax.axis_index("c")
    sid = lax.axis_index("s")
    wid = sid * 2 + cid

    @pl.when(wid < n_tasks)
    def _():
        pltpu.sync_copy(wts, wtbuf)
        z16 = jnp.zeros((16,), jnp.float32)
        for b in range(8):
            mxbuf[b, pl.ds(992, 16)] = z16
            mxbuf[b, pl.ds(1008, 16)] = z16
        for rr in range(16):
            refbuf[rr, pl.ds(992, 16)] = z16
            refbuf[rr, pl.ds(1008, 16)] = z16
        t0 = wid * SC_WG
        ridx = lax.iota(jnp.int32, 16)
        lanes = lax.iota(jnp.int32, 16)

        def pair_step(pair, carry):
            col0 = (t0 + 2 * pair) * WIN
            pltpu.sync_copy(mx_hbm.at[:, pl.ds(col0, 2 * WIN)],
                            mxbuf.at[:, pl.ds(0, 2 * WIN)])
            pltpu.sync_copy(r_hbm.at[:, pl.ds(col0, 2 * WIN)],
                            refbuf.at[:, pl.ds(0, 2 * WIN)])
            for bb in range(4):
                b0, b1 = 2 * bb, 2 * bb + 1
                for half in range(2):
                    off = half * 496
                    lo, hi = half * WIN, (half + 1) * WIN

                    def chunk_step(i, accs):
                        base = off + i * 16
                        pos = base + lanes
                        maskf = jnp.where(
                            (pos >= lo) & (pos < hi), 1.0, 0.0
                        ).astype(jnp.float32)
                        m0 = mxbuf[b0, pl.ds(base, 16)] * maskf
                        m1 = mxbuf[b1, pl.ds(base, 16)] * maskf
                        acc0 = list(accs[:16])
                        acc1 = list(accs[16:])
                        for rr in range(16):
                            rc = refbuf[rr, pl.ds(base, 16)]
                            acc0[rr] = acc0[rr] + m0 * rc
                            acc1[rr] = acc1[rr] + m1 * rc
                        return tuple(acc0) + tuple(acc1)

                    init = (jnp.zeros((16,), jnp.float32),) * 32
                    accs = lax.fori_loop(0, SC_CHUNKS, chunk_step, init)
                    for j in range(2):
                        for rr in range(16):
                            accbuf[j, rr, :] = accs[16 * j + rr]
                    w = 2 * pair + half
                    for j, b in ((0, b0), (1, b1)):
                        vsum = jnp.zeros((16,), jnp.float32)
                        for l in range(16):
                            col = plsc.load_gather(
                                accbuf.at[j],
                                [ridx, jnp.full((16,), l, jnp.int32)],
                            )
                            vsum = vsum + col
                        wtbl[b, w, :] = vsum * (1.0 / WIN)
            return carry

        lax.fori_loop(0, SC_WG // 2, pair_step, 0)

        wt0 = wtbuf[0, :]
        wt1 = wtbuf[1, :]
        widx = lax.iota(jnp.int32, 16)
        neg = jnp.full((16,), NEG_INF, jnp.float32)
        zero_i = jnp.zeros((16,), jnp.int32)
        for b in range(8):
            best = plsc.load_gather(wtbl.at[b], [widx, zero_i])
            bidx = zero_i
            sec = neg
            sidx = zero_i
            for rr in range(1, 16):
                v = plsc.load_gather(
                    wtbl.at[b], [widx, jnp.full((16,), rr, jnp.int32)]
                )
                rvec = jnp.full((16,), rr, jnp.int32)
                c1 = v > best
                c2 = v > sec
                sec = jnp.where(c1, best, jnp.where(c2, v, sec))
                sidx = jnp.where(c1, bidx, jnp.where(c2, rvec, sidx))
                best = jnp.where(c1, v, best)
                bidx = jnp.where(c1, rvec, bidx)
            ovbuf[b, :] = best * wt0 + sec * wt1
            ox1buf[b, :] = bidx
            ox2buf[b, :] = sidx

        pltpu.sync_copy(ovbuf.at[:, pl.ds(0, SC_WG)],
                        po.at[:, pl.ds(t0, SC_WG)])
        pltpu.sync_copy(ox1buf.at[:, pl.ds(0, SC_WG)],
                        io.at[:, 0, pl.ds(t0, SC_WG)])
        pltpu.sync_copy(ox2buf.at[:, pl.ds(0, SC_WG)],
                        io.at[:, 1, pl.ds(t0, SC_WG)])


def _sc_panel(mixed, ref, weights, bs, n_windows):
    mesh = plsc.VectorSubcoreMesh(
        core_axis_name="c", subcore_axis_name="s", num_cores=2, num_subcores=16
    )
    out_type = [
        jax.ShapeDtypeStruct((bs, n_windows), jnp.float32),
        jax.ShapeDtypeStruct((bs, K, n_windows), jnp.int32),
    ]
    scratch = [
        pltpu.VMEM((8, 1024), jnp.float32),
        pltpu.VMEM((16, 1024), jnp.float32),
        pltpu.VMEM((2, 16, 16), jnp.float32),
        pltpu.VMEM((8, 16, 16), jnp.float32),
        pltpu.VMEM((K, 16), jnp.float32),
        pltpu.VMEM((8, 16), jnp.float32),
        pltpu.VMEM((8, 16), jnp.int32),
        pltpu.VMEM((8, 16), jnp.int32),
    ]
    body = functools.partial(_sc_dense_body, n_windows)
    fn = pl.kernel(
        body,
        out_type=out_type,
        mesh=mesh,
        scratch_types=scratch,
        compiler_params=pltpu.CompilerParams(
            needs_layout_passes=False, use_tc_tiling_on_sc=False
        ),
    )
    wts16 = jnp.broadcast_to(weights[:K], (K, 16))
    return fn(mixed, ref, wts16)


def kernel(input_mixed, ref_panel_0, ref_panel_1, ref_panel_2, weights):
    bs, n_snps = input_mixed.shape
    n_refs = ref_panel_0.shape[0]
    n_windows = n_snps // WIN
    p0, i0, p1, i1 = _tc_panels(
        input_mixed, ref_panel_0, ref_panel_1, weights, bs, n_refs, n_windows
    )
    p2, i2 = _sc_panel(input_mixed, ref_panel_2, weights, bs, n_windows)
    return (p0, p1, p2, i0, i1, i2)


# speedup vs baseline: 1.6086x; 1.0330x over previous
import functools

import jax
import jax.numpy as jnp
from jax import lax
from jax.experimental import pallas as pl
from jax.experimental.pallas import tpu as pltpu
from jax.experimental.pallas import tpu_sc as plsc

WIN = 500
K = 2
TC_WB = 32
SC_WG = 8
SC_CHUNKS = 33
SC_SPAN = 1152

NEG_INF = float("-inf")


def _tc_fused_body(wt_ref, mx_ref, ra_ref, rb_ref,
                   pa_ref, ia_ref, pb_ref, ib_ref):
    mx = mx_ref[...]
    inv = 1.0 / WIN
    wt0 = wt_ref[0, 0]
    wt1 = wt_ref[1, 0]
    for r_ref, p_ref, idx_ref in ((ra_ref, pa_ref, ia_ref),
                                  (rb_ref, pb_ref, ib_ref)):
        r = r_ref[...]
        ms = []
        for j in range(TC_WB):
            a = lax.slice(mx, (0, j * WIN), (8, (j + 1) * WIN))
            b = lax.slice(r, (0, j * WIN), (16, (j + 1) * WIN))
            m = lax.dot_general(
                a, b,
                dimension_numbers=(((1,), (1,)), ((), ())),
                preferred_element_type=jnp.float32,
                precision=lax.Precision.HIGHEST,
            )
            ms.append(m * inv)
        M = jnp.stack(ms, axis=0)
        li = lax.broadcasted_iota(jnp.int32, M.shape, 2)
        max1 = jnp.max(M, axis=-1)
        i1 = jnp.min(jnp.where(M == max1[..., None], li, 127), axis=-1)
        M2 = jnp.where(li == i1[..., None], NEG_INF, M)
        max2 = jnp.max(M2, axis=-1)
        i2 = jnp.min(jnp.where(M2 == max2[..., None], li, 127), axis=-1)
        p_ref[...] = max1 * wt0 + max2 * wt1
        idx_ref[...] = jnp.stack([i1, i2], axis=1)


def _tc_panels(mixed, ref_a, ref_b, weights, bs, n_refs, n_windows):
    grid = -(-n_windows // TC_WB)
    nw_pad = grid * TC_WB
    cols = TC_WB * WIN
    out_shape = [
        jax.ShapeDtypeStruct((nw_pad, bs), jnp.float32),
        jax.ShapeDtypeStruct((nw_pad, K, bs), jnp.int32),
    ] * 2
    in_specs = [
        pl.BlockSpec(memory_space=pltpu.SMEM),
        pl.BlockSpec((bs, cols), lambda i: (0, i)),
        pl.BlockSpec((n_refs, cols), lambda i: (0, i)),
        pl.BlockSpec((n_refs, cols), lambda i: (0, i)),
    ]
    out_specs = [
        pl.BlockSpec((TC_WB, bs), lambda i: (i, 0)),
        pl.BlockSpec((TC_WB, K, bs), lambda i: (i, 0, 0)),
    ] * 2
    pa, ia, pb, ib = pl.pallas_call(
        _tc_fused_body,
        grid=(grid,),
        in_specs=in_specs,
        out_specs=out_specs,
        out_shape=out_shape,
    )(weights, mixed, ref_a, ref_b)
    outs = []
    for p_t, i_t in ((pa, ia), (pb, ib)):
        outs.append(p_t[:n_windows].T)
        outs.append(jnp.transpose(i_t[:n_windows], (2, 1, 0)))
    return outs


def _sc_dense_body(nw, mx_hbm, r_hbm, wts, po, i1o, i2o,
                   mxbuf, refbuf, accbuf, wtbl, wtbuf, opbuf, ob1buf, ob2buf):
    n_tasks = nw // SC_WG
    cid = lax.axis_index("c")
    sid = lax.axis_index("s")
    wid = sid * 2 + cid

    @pl.when(wid < n_tasks)
    def _():
        pltpu.sync_copy(wts, wtbuf)
        t0 = pl.multiple_of(wid * SC_WG, 8)
        ridx = lax.iota(jnp.int32, 16)
        lanes = lax.iota(jnp.int32, 16)

        def pair_step(pair, carry):
            col0 = (t0 + 2 * pair) * WIN
            c_lo = pl.multiple_of((col0 // 128) * 128, 128)
            delta = pl.multiple_of(col0 - c_lo, 8)
            pltpu.sync_copy(mx_hbm.at[:, pl.ds(c_lo, SC_SPAN)], mxbuf)
            pltpu.sync_copy(r_hbm.at[:, pl.ds(c_lo, SC_SPAN)], refbuf)
            for bb in range(4):
                b0, b1 = 2 * bb, 2 * bb + 1
                for half in range(2):
                    lo = delta + half * WIN
                    hi = lo + WIN
                    off0 = pl.multiple_of((lo // 16) * 16, 16)

                    def chunk_step(i, accs):
                        base = off0 + i * 16
                        pos = base + lanes
                        maskf = jnp.where(
                            (pos >= lo) & (pos < hi), 1.0, 0.0
                        ).astype(jnp.float32)
                        m0 = mxbuf[b0, pl.ds(base, 16)] * maskf
                        m1 = mxbuf[b1, pl.ds(base, 16)] * maskf
                        acc0 = list(accs[:16])
                        acc1 = list(accs[16:])
                        for rr in range(16):
                            rc = refbuf[rr, pl.ds(base, 16)]
                            acc0[rr] = acc0[rr] + m0 * rc
                            acc1[rr] = acc1[rr] + m1 * rc
                        return tuple(acc0) + tuple(acc1)

                    init = (jnp.zeros((16,), jnp.float32),) * 32
                    accs = lax.fori_loop(0, SC_CHUNKS, chunk_step, init)
                    for j in range(2):
                        for rr in range(16):
                            accbuf[j, rr, :] = accs[16 * j + rr]
                    w = 2 * pair + half
                    for j, b in ((0, b0), (1, b1)):
                        vsum = jnp.zeros((16,), jnp.float32)
                        for l in range(16):
                            col = plsc.load_gather(
                                accbuf.at[j],
                                [ridx, jnp.full((16,), l, jnp.int32)],
                            )
                            vsum = vsum + col
                        wtbl[b, w, :] = vsum * (1.0 / WIN)
            return carry

        lax.fori_loop(0, SC_WG // 2, pair_step, 0)

        wt0 = wtbuf[0, :]
        wt1 = wtbuf[1, :]
        widx = lax.iota(jnp.int32, 16)
        neg = jnp.full((16,), NEG_INF, jnp.float32)
        zero_i = jnp.zeros((16,), jnp.int32)
        for b in range(8):
            best = plsc.load_gather(wtbl.at[b], [widx, zero_i])
            bidx = zero_i
            sec = neg
            sidx = zero_i
            for rr in range(1, 16):
                v = plsc.load_gather(
                    wtbl.at[b], [widx, jnp.full((16,), rr, jnp.int32)]
                )
                rvec = jnp.full((16,), rr, jnp.int32)
                c1 = v > best
                c2 = v > sec
                sec = jnp.where(c1, best, jnp.where(c2, v, sec))
                sidx = jnp.where(c1, bidx, jnp.where(c2, rvec, sidx))
                best = jnp.where(c1, v, best)
                bidx = jnp.where(c1, rvec, bidx)
            bvec = jnp.full((16,), b, jnp.int32)
            plsc.store_scatter(opbuf, [widx, bvec], best * wt0 + sec * wt1)
            plsc.store_scatter(ob1buf, [widx, bvec], bidx)
            plsc.store_scatter(ob2buf, [widx, bvec], sidx)

        pltpu.sync_copy(opbuf.at[pl.ds(0, SC_WG), :],
                        po.at[pl.ds(t0, SC_WG), :])
        pltpu.sync_copy(ob1buf.at[pl.ds(0, SC_WG), :],
                        i1o.at[pl.ds(t0, SC_WG), :])
        pltpu.sync_copy(ob2buf.at[pl.ds(0, SC_WG), :],
                        i2o.at[pl.ds(t0, SC_WG), :])


def _sc_panel(mixed, ref, weights, bs, n_windows):
    mesh = plsc.VectorSubcoreMesh(
        core_axis_name="c", subcore_axis_name="s", num_cores=2, num_subcores=16
    )
    out_type = [
        jax.ShapeDtypeStruct((n_windows, bs), jnp.float32),
        jax.ShapeDtypeStruct((n_windows, bs), jnp.int32),
        jax.ShapeDtypeStruct((n_windows, bs), jnp.int32),
    ]
    scratch = [
        pltpu.VMEM((8, SC_SPAN), jnp.float32),
        pltpu.VMEM((16, SC_SPAN), jnp.float32),
        pltpu.VMEM((2, 16, 16), jnp.float32),
        pltpu.VMEM((8, 16, 16), jnp.float32),
        pltpu.VMEM((K, 16), jnp.float32),
        pltpu.VMEM((16, 8), jnp.float32),
        pltpu.VMEM((16, 8), jnp.int32),
        pltpu.VMEM((16, 8), jnp.int32),
    ]
    body = functools.partial(_sc_dense_body, n_windows)
    fn = pl.kernel(
        body,
        out_type=out_type,
        mesh=mesh,
        scratch_types=scratch,
        compiler_params=pltpu.CompilerParams(
            needs_layout_passes=False, use_tc_tiling_on_sc=True
        ),
    )
    wts16 = jnp.broadcast_to(weights[:K], (K, 16))
    p_t, i1_t, i2_t = fn(mixed, ref, wts16)
    pooled = p_t.T
    idx = jnp.stack([i1_t.T, i2_t.T], axis=1)
    return pooled, idx


def kernel(input_mixed, ref_panel_0, ref_panel_1, ref_panel_2, weights):
    bs, n_snps = input_mixed.shape
    n_refs = ref_panel_0.shape[0]
    n_windows = n_snps // WIN
    p0, i0, p1, i1 = _tc_panels(
        input_mixed, ref_panel_0, ref_panel_1, weights, bs, n_refs, n_windows
    )
    p2, i2 = _sc_panel(input_mixed, ref_panel_2, weights, bs, n_windows)
    return (p0, p1, p2, i0, i1, i2)
